# baseline (speedup 1.0000x reference)
import functools

import jax
import jax.numpy as jnp
from jax import lax
from jax.experimental import pallas as pl
from jax.experimental.pallas import tpu as pltpu
from jax.experimental.pallas import tpu_sc as plsc

N_VARIANTXGENES = 1_000_000
B, S = 16384, 50
D = 64
N_ROWS = B * S
CHUNK = 128
NW = 32
CHUNKS_PER_W = N_ROWS // (CHUNK * NW)
NB = 8


def _sc_gather(idx2d, table):
    mesh = plsc.VectorSubcoreMesh(core_axis_name="c", subcore_axis_name="s")

    @functools.partial(
        pl.kernel,
        out_type=jax.ShapeDtypeStruct((N_ROWS, D), jnp.float32),
        mesh=mesh,
        scratch_types=[
            pltpu.VMEM((CHUNKS_PER_W, CHUNK), jnp.int32),
            pltpu.VMEM((NB, CHUNK, D), jnp.float32),
            pltpu.SemaphoreType.DMA((NB,)),
            pltpu.SemaphoreType.DMA((NB,)),
        ],
        compiler_params=pltpu.CompilerParams(use_tc_tiling_on_sc=False),
    )
    def k(idx_hbm, tbl, out_hbm, idx_v, gbuf, gsem, ssem):
        wid = lax.axis_index("s") * 2 + lax.axis_index("c")
        pltpu.sync_copy(idx_hbm.at[pl.ds(wid * CHUNKS_PER_W, CHUNKS_PER_W)], idx_v)

        def fire(b, j):
            pltpu.async_copy(tbl.at[idx_v.at[j]], gbuf.at[b], gsem.at[b])

        for b in range(NB):
            fire(b, b)

        def group(g, carry):
            for b in range(NB):
                j = g * NB + b
                pltpu.make_async_copy(
                    tbl.at[idx_v.at[0]], gbuf.at[b], gsem.at[b]
                ).wait()
                base = (wid * CHUNKS_PER_W + j) * CHUNK
                pltpu.async_copy(
                    gbuf.at[b], out_hbm.at[pl.ds(base, CHUNK)], ssem.at[b]
                )
                nxt = j + NB

                @pl.when(nxt < CHUNKS_PER_W)
                def _():
                    pltpu.make_async_copy(
                        gbuf.at[b], out_hbm.at[pl.ds(0, CHUNK)], ssem.at[b]
                    ).wait()
                    fire(b, nxt)

            return carry

        lax.fori_loop(0, CHUNKS_PER_W // NB, group, 0)

        for b in range(NB):
            pltpu.make_async_copy(
                gbuf.at[b], out_hbm.at[pl.ds(0, CHUNK)], ssem.at[b]
            ).wait()

    return k(idx2d, table)


def kernel(variantxgene_ixs, table):
    idx2d = variantxgene_ixs.reshape(N_ROWS // CHUNK, CHUNK).astype(jnp.int32)
    out = _sc_gather(idx2d, table)
    return out.reshape(B, S, D)
